# baseline (speedup 1.0000x reference)
import functools

import jax
import jax.numpy as jnp
from jax import lax
from jax.experimental import pallas as pl
from jax.experimental.pallas import tpu as pltpu
from jax.experimental.pallas import tpu_sc as plsc

NC = 2
NS = 16
NW = NC * NS
LANE = 16
CHUNK = 40


@functools.lru_cache(maxsize=None)
def _make_pool(B, L, D, V):
    assert B % NW == 0 and L % CHUNK == 0 and D % LANE == 0
    BPW = B // NW
    NV = D // LANE
    NCH = L // CHUNK
    mesh = plsc.VectorSubcoreMesh(core_axis_name="c", subcore_axis_name="s")

    @functools.partial(
        pl.kernel,
        mesh=mesh,
        out_type=jax.ShapeDtypeStruct((NW, BPW, D), jnp.float32),
        compiler_params=pltpu.CompilerParams(use_tc_tiling_on_sc=False),
        scratch_types=[
            pltpu.VMEM((BPW * L,), jnp.int32),
            pltpu.VMEM((2, L, D), jnp.float32),
            pltpu.VMEM((BPW, D), jnp.float32),
            pltpu.SemaphoreType.DMA,
            pltpu.SemaphoreType.DMA,
        ],
    )
    def pool(table_hbm, x_hbm, out_hbm, idx_v, buf_v, acc_v, s0, s1):
        wid = lax.axis_index("s") * NC + lax.axis_index("c")
        sems = (s0, s1)

        pltpu.sync_copy(x_hbm.at[wid], idx_v)

        def row_dmas(row, pair):
            return [
                pltpu.make_async_copy(
                    table_hbm.at[idx_v.at[pl.ds(row * L + k * CHUNK, CHUNK)]],
                    buf_v.at[pair, pl.ds(k * CHUNK, CHUNK), :],
                    sems[pair],
                )
                for k in range(NCH)
            ]

        def gather(row, pair):
            for dma in row_dmas(row, pair):
                dma.start()

        def wait(row, pair):
            for dma in row_dmas(row, pair):
                dma.wait()

        gather(0, 0)
        gather(1, 1)

        def outer(rr, carry):
            for pair in range(2):
                row = rr * 2 + pair
                wait(row, pair)

                def inner(i, acc):
                    new = []
                    for d in range(NV):
                        a = acc[d]
                        for u in range(2):
                            a = a + buf_v[pair, i * 2 + u, pl.ds(d * LANE, LANE)]
                        new.append(a)
                    return tuple(new)

                zeros = tuple(jnp.zeros((LANE,), jnp.float32) for _ in range(NV))
                acc = lax.fori_loop(0, L // 2, inner, zeros)
                for d in range(NV):
                    acc_v[row, pl.ds(d * LANE, LANE)] = acc[d]

                @pl.when(row + 2 < BPW)
                def _():
                    gather(row + 2, pair)

            return carry

        lax.fori_loop(0, BPW // 2, outer, 0)
        pltpu.sync_copy(acc_v, out_hbm.at[wid])

    return pool


@functools.lru_cache(maxsize=None)
def _make_tr(V, D, BV):
    def body(t_ref, o_ref):
        eye = jax.lax.broadcasted_iota(jnp.int32, (D, D), 0) == jax.lax.broadcasted_iota(jnp.int32, (D, D), 1)
        o_ref[:, 0:D] = lax.dot_general(
            t_ref[...],
            eye.astype(jnp.float32),
            (((0,), (0,)), ((), ())),
            preferred_element_type=jnp.float32,
        )

    return pl.pallas_call(
        body,
        grid=(pl.cdiv(V, BV),),
        in_specs=[pl.BlockSpec((D, BV), lambda j: (0, j))],
        out_specs=pl.BlockSpec((BV, 2 * D), lambda j: (j, 0)),
        out_shape=jax.ShapeDtypeStruct((V, 2 * D), jnp.float32),
    )


@functools.lru_cache(maxsize=None)
def _make_proj(B, D, H):
    def body(s_ref, w_ref, b_ref, o_ref):
        o_ref[...] = (
            lax.dot_general(
                s_ref[...],
                w_ref[...],
                (((1,), (1,)), ((), ())),
                preferred_element_type=jnp.float32,
            )
            + b_ref[...]
        )

    return pl.pallas_call(
        body,
        out_shape=jax.ShapeDtypeStruct((B, H), jnp.float32),
    )


def kernel(x, table, W, b):
    B, L = x.shape
    V, D = table.shape
    H = W.shape[0]
    xi = (x.astype(jnp.int32) * 2).reshape(NW, (B // NW) * L)
    tpad = _make_tr(V, D, 4096)(table.T)
    s = _make_pool(B, L, D, V)(tpad.reshape(2 * V, D), xi)
    s = s.reshape(B, D)
    return _make_proj(B, D, H)(s, W, b.reshape(1, H))


# speedup vs baseline: 1.2030x; 1.2030x over previous
import functools

import jax
import jax.numpy as jnp
from jax import lax
from jax.experimental import pallas as pl
from jax.experimental.pallas import tpu as pltpu
from jax.experimental.pallas import tpu_sc as plsc

NC = 2
NS = 16
NW = NC * NS
LANE = 16
CHUNK = 40


@functools.lru_cache(maxsize=None)
def _make_pool(B, L, D, V):
    assert B % NW == 0 and L % CHUNK == 0 and D % LANE == 0
    BPW = B // NW
    NV = D // LANE
    NCH = L // CHUNK
    mesh = plsc.VectorSubcoreMesh(core_axis_name="c", subcore_axis_name="s")

    @functools.partial(
        pl.kernel,
        mesh=mesh,
        out_type=jax.ShapeDtypeStruct((NW, BPW, D), jnp.float32),
        compiler_params=pltpu.CompilerParams(use_tc_tiling_on_sc=False),
        scratch_types=[
            pltpu.VMEM((BPW * L,), jnp.int32),
            pltpu.VMEM((2, L, D), jnp.float32),
            pltpu.VMEM((BPW, D), jnp.float32),
            pltpu.SemaphoreType.DMA,
            pltpu.SemaphoreType.DMA,
        ],
    )
    def pool(table_hbm, x_hbm, out_hbm, idx_v, buf_v, acc_v, s0, s1):
        wid = lax.axis_index("s") * NC + lax.axis_index("c")
        sems = (s0, s1)

        pltpu.sync_copy(x_hbm.at[wid], idx_v)

        def row_dmas(row, pair):
            return [
                pltpu.make_async_copy(
                    table_hbm.at[idx_v.at[pl.ds(row * L + k * CHUNK, CHUNK)]],
                    buf_v.at[pair, pl.ds(k * CHUNK, CHUNK), :],
                    sems[pair],
                )
                for k in range(NCH)
            ]

        def gather(row, pair):
            for dma in row_dmas(row, pair):
                dma.start()

        def wait(row, pair):
            for dma in row_dmas(row, pair):
                dma.wait()

        gather(0, 0)
        gather(1, 1)

        def outer(rr, carry):
            for pair in range(2):
                row = rr * 2 + pair
                wait(row, pair)

                def inner(i, acc):
                    new = []
                    for d in range(NV):
                        a = acc[d]
                        for u in range(2):
                            a = a + buf_v[pair, i * 2 + u, pl.ds(d * LANE, LANE)]
                        new.append(a)
                    return tuple(new)

                zeros = tuple(jnp.zeros((LANE,), jnp.float32) for _ in range(NV))
                acc = lax.fori_loop(0, L // 2, inner, zeros)
                for d in range(NV):
                    acc_v[row, pl.ds(d * LANE, LANE)] = acc[d]

                @pl.when(row + 2 < BPW)
                def _():
                    gather(row + 2, pair)

            return carry

        lax.fori_loop(0, BPW // 2, outer, 0)
        pltpu.sync_copy(acc_v, out_hbm.at[wid])

    return pool


@functools.lru_cache(maxsize=None)
def _make_tr(V, D, BV):
    def body(t_ref, o_ref):
        o_ref[:, 0:D] = t_ref[...].T

    return pl.pallas_call(
        body,
        grid=(pl.cdiv(V, BV),),
        in_specs=[pl.BlockSpec((D, BV), lambda j: (0, j))],
        out_specs=pl.BlockSpec((BV, 2 * D), lambda j: (j, 0)),
        out_shape=jax.ShapeDtypeStruct((V, 2 * D), jnp.float32),
    )


@functools.lru_cache(maxsize=None)
def _make_proj(B, D, H):
    def body(s_ref, w_ref, b_ref, o_ref):
        o_ref[...] = (
            lax.dot_general(
                s_ref[...],
                w_ref[...],
                (((1,), (1,)), ((), ())),
                preferred_element_type=jnp.float32,
            )
            + b_ref[...]
        )

    return pl.pallas_call(
        body,
        out_shape=jax.ShapeDtypeStruct((B, H), jnp.float32),
    )


def kernel(x, table, W, b):
    B, L = x.shape
    V, D = table.shape
    H = W.shape[0]
    xi = (x.astype(jnp.int32) * 2).reshape(NW, (B // NW) * L)
    tpad = _make_tr(V, D, 8192)(table.T)
    s = _make_pool(B, L, D, V)(tpad.reshape(2 * V, D), xi)
    s = s.reshape(B, D)
    return _make_proj(B, D, H)(s, W, b.reshape(1, H))


# speedup vs baseline: 1.2644x; 1.0510x over previous
import functools

import jax
import jax.numpy as jnp
from jax import lax
from jax.experimental import pallas as pl
from jax.experimental.pallas import tpu as pltpu
from jax.experimental.pallas import tpu_sc as plsc

NC = 2
NS = 16
NW = NC * NS
LANE = 16
CHUNK = 40


@functools.lru_cache(maxsize=None)
def _make_pool(B, L, D, V):
    assert B % NW == 0 and L % CHUNK == 0 and D % LANE == 0
    BPW = B // NW
    NV = D // LANE
    NCH = L // CHUNK
    mesh = plsc.VectorSubcoreMesh(core_axis_name="c", subcore_axis_name="s")

    @functools.partial(
        pl.kernel,
        mesh=mesh,
        out_type=jax.ShapeDtypeStruct((NW, BPW, D), jnp.float32),
        compiler_params=pltpu.CompilerParams(use_tc_tiling_on_sc=False),
        scratch_types=[
            pltpu.VMEM((BPW * L,), jnp.int32),
            pltpu.VMEM((2, L, D), jnp.float32),
            pltpu.VMEM((BPW, D), jnp.float32),
            pltpu.SemaphoreType.DMA,
            pltpu.SemaphoreType.DMA,
        ],
    )
    def pool(table_hbm, x_hbm, out_hbm, idx_v, buf_v, acc_v, s0, s1):
        wid = lax.axis_index("s") * NC + lax.axis_index("c")
        sems = (s0, s1)

        pltpu.sync_copy(x_hbm.at[wid], idx_v)

        def row_dmas(row, pair):
            return [
                pltpu.make_async_copy(
                    table_hbm.at[idx_v.at[pl.ds(row * L + k * CHUNK, CHUNK)]],
                    buf_v.at[pair, pl.ds(k * CHUNK, CHUNK), :],
                    sems[pair],
                )
                for k in range(NCH)
            ]

        def gather(row, pair):
            for dma in row_dmas(row, pair):
                dma.start()

        def wait(row, pair):
            for dma in row_dmas(row, pair):
                dma.wait()

        gather(0, 0)
        gather(1, 1)

        def outer(rr, carry):
            for pair in range(2):
                row = rr * 2 + pair
                wait(row, pair)

                def inner(i, acc):
                    new = []
                    for d in range(NV):
                        a = acc[d]
                        for u in range(2):
                            a = a + buf_v[pair, i * 2 + u, pl.ds(d * LANE, LANE)]
                        new.append(a)
                    return tuple(new)

                zeros = tuple(jnp.zeros((LANE,), jnp.float32) for _ in range(NV))
                acc = lax.fori_loop(0, L // 2, inner, zeros)
                for d in range(NV):
                    acc_v[row, pl.ds(d * LANE, LANE)] = acc[d]

                @pl.when(row + 2 < BPW)
                def _():
                    gather(row + 2, pair)

            return carry

        lax.fori_loop(0, BPW // 2, outer, 0)
        pltpu.sync_copy(acc_v, out_hbm.at[wid])

    return pool


@functools.lru_cache(maxsize=None)
def _make_tr(V, D, BV):
    def body(t_ref, o_ref):
        o_ref[:, 0:D] = t_ref[...].T

    return pl.pallas_call(
        body,
        grid=(pl.cdiv(V, BV),),
        in_specs=[pl.BlockSpec((D, BV), lambda j: (0, j))],
        out_specs=pl.BlockSpec((BV, 2 * D), lambda j: (j, 0)),
        out_shape=jax.ShapeDtypeStruct((V, 2 * D), jnp.float32),
    )


@functools.lru_cache(maxsize=None)
def _make_proj(B, D, H):
    def body(s_ref, w_ref, b_ref, o_ref):
        o_ref[...] = (
            lax.dot_general(
                s_ref[...],
                w_ref[...],
                (((1,), (1,)), ((), ())),
                preferred_element_type=jnp.float32,
            )
            + b_ref[...]
        )

    return pl.pallas_call(
        body,
        out_shape=jax.ShapeDtypeStruct((B, H), jnp.float32),
    )


def kernel(x, table, W, b):
    B, L = x.shape
    V, D = table.shape
    H = W.shape[0]
    xi = (x.astype(jnp.int32) * 2).reshape(NW, (B // NW) * L)
    tpad = _make_tr(V, D, 16384)(table.T)
    s = _make_pool(B, L, D, V)(tpad.reshape(2 * V, D), xi)
    s = s.reshape(B, D)
    return _make_proj(B, D, H)(s, W, b.reshape(1, H))


# speedup vs baseline: 1.2818x; 1.0138x over previous
import functools

import jax
import jax.numpy as jnp
from jax import lax
from jax.experimental import pallas as pl
from jax.experimental.pallas import tpu as pltpu
from jax.experimental.pallas import tpu_sc as plsc

NC = 2
NS = 16
NW = NC * NS
LANE = 16
CHUNK = 40


@functools.lru_cache(maxsize=None)
def _make_pool(B, L, D, V):
    assert B % NW == 0 and L % CHUNK == 0 and D % LANE == 0
    BPW = B // NW
    NV = D // LANE
    NCH = L // CHUNK
    mesh = plsc.VectorSubcoreMesh(core_axis_name="c", subcore_axis_name="s")

    @functools.partial(
        pl.kernel,
        mesh=mesh,
        out_type=jax.ShapeDtypeStruct((NW, BPW, D), jnp.float32),
        compiler_params=pltpu.CompilerParams(use_tc_tiling_on_sc=False),
        scratch_types=[
            pltpu.VMEM((BPW * L,), jnp.int32),
            pltpu.VMEM((2, L, D), jnp.float32),
            pltpu.VMEM((BPW, D), jnp.float32),
            pltpu.SemaphoreType.DMA,
            pltpu.SemaphoreType.DMA,
        ],
    )
    def pool(table_hbm, x_hbm, out_hbm, idx_v, buf_v, acc_v, s0, s1):
        wid = lax.axis_index("s") * NC + lax.axis_index("c")
        sems = (s0, s1)

        pltpu.sync_copy(x_hbm.at[wid], idx_v)

        def row_dmas(row, pair):
            return [
                pltpu.make_async_copy(
                    table_hbm.at[idx_v.at[pl.ds(row * L + k * CHUNK, CHUNK)]],
                    buf_v.at[pair, pl.ds(k * CHUNK, CHUNK), :],
                    sems[pair],
                )
                for k in range(NCH)
            ]

        def gather(row, pair):
            for dma in row_dmas(row, pair):
                dma.start()

        def wait(row, pair):
            for dma in row_dmas(row, pair):
                dma.wait()

        gather(0, 0)
        gather(1, 1)

        def outer(rr, carry):
            for pair in range(2):
                row = rr * 2 + pair
                wait(row, pair)

                def inner(i, acc):
                    new = []
                    for d in range(NV):
                        a = acc[d]
                        for u in range(2):
                            a = a + buf_v[pair, i * 2 + u, pl.ds(d * LANE, LANE)]
                        new.append(a)
                    return tuple(new)

                zeros = tuple(jnp.zeros((LANE,), jnp.float32) for _ in range(NV))
                acc = lax.fori_loop(0, L // 2, inner, zeros)
                for d in range(NV):
                    acc_v[row, pl.ds(d * LANE, LANE)] = acc[d]

                @pl.when(row + 2 < BPW)
                def _():
                    gather(row + 2, pair)

            return carry

        lax.fori_loop(0, BPW // 2, outer, 0)
        pltpu.sync_copy(acc_v, out_hbm.at[wid])

    return pool


@functools.lru_cache(maxsize=None)
def _make_tr(V, D, BV):
    def body(t_ref, o_ref):
        o_ref[:, 0:D] = t_ref[...].T

    return pl.pallas_call(
        body,
        grid=(pl.cdiv(V, BV),),
        in_specs=[pl.BlockSpec((D, BV), lambda j: (0, j))],
        out_specs=pl.BlockSpec((BV, 2 * D), lambda j: (j, 0)),
        out_shape=jax.ShapeDtypeStruct((V, 2 * D), jnp.float32),
    )


@functools.lru_cache(maxsize=None)
def _make_proj(B, D, H):
    def body(s_ref, w_ref, b_ref, o_ref):
        o_ref[...] = (
            lax.dot_general(
                s_ref[...],
                w_ref[...],
                (((1,), (1,)), ((), ())),
                preferred_element_type=jnp.float32,
            )
            + b_ref[...]
        )

    return pl.pallas_call(
        body,
        out_shape=jax.ShapeDtypeStruct((B, H), jnp.float32),
    )


def kernel(x, table, W, b):
    B, L = x.shape
    V, D = table.shape
    H = W.shape[0]
    xi = (x.astype(jnp.int32) * 2).reshape(NW, (B // NW) * L)
    tpad = _make_tr(V, D, 32768)(table.T)
    s = _make_pool(B, L, D, V)(tpad.reshape(2 * V, D), xi)
    s = s.reshape(B, D)
    return _make_proj(B, D, H)(s, W, b.reshape(1, H))
